# baseline (speedup 1.0000x reference)
import functools

import jax
import jax.numpy as jnp
from jax import lax
from jax.experimental import pallas as pl
from jax.experimental.pallas import tpu as pltpu
from jax.experimental.pallas import tpu_sc as plsc

N = 10000
E = 160000
D = 300
DE = 16
L = 5
NL = 2000
NU = 256

DP = 320
DH = DP // 2
NSUB = 16
CH = 64
CPW = 160
E_PAD = NSUB * CPW * CH
N_ACC = 10016
N_OUT = N_ACC
BN = 400
BNP = 1000
BE = 2048


def _edge_proj_kernel(ea_ref, ew_ref, eb_ref, o_ref):
    v = jnp.dot(ea_ref[...], ew_ref[0], preferred_element_type=jnp.float32)
    v = v + eb_ref[0]
    o_ref[0, 0, :, 0:128] = v[:, 0:128]
    o_ref[0, 0, :, 128:256] = v[:, 32:160]
    o_ref[0, 1, :, 0:128] = v[:, 160:288]
    o_ref[0, 1, :, 128:256] = v[:, 192:320]


def _dense_kernel(a_ref, w_ref, b_ref, o_ref):
    a = jnp.concatenate([a_ref[0], a_ref[1]], axis=1)
    v = jnp.dot(a, w_ref[...], preferred_element_type=jnp.float32) + b_ref[...]
    v = jnp.maximum(v, 0.0)
    o_ref[0] = v[:, :DH]
    o_ref[1] = v[:, DH:]


def _lower_pool_kernel(lb_ref, h_ref, o_ref):
    i = pl.program_id(0)
    lb = lb_ref[0, 0]
    h = jnp.concatenate([h_ref[0], h_ref[1]], axis=1)
    haug = jnp.concatenate([h, jnp.ones((BNP, 8), jnp.float32)], axis=1)
    gi = lax.broadcasted_iota(jnp.int32, (NL, BNP), 0)
    ind = (gi == lb[None, :]).astype(jnp.float32)
    part = jnp.dot(ind, haug, preferred_element_type=jnp.float32)

    @pl.when(i == 0)
    def _():
        o_ref[...] = part

    @pl.when(i != 0)
    def _():
        o_ref[...] = o_ref[...] + part


def _final_kernel(p_ref, ub_ref, ub2_ref, c1_ref, c1b_ref, c2_ref, c2b_ref,
                  o_ref):
    pooled = p_ref[...]
    cnt = jnp.clip(pooled[:, DP:DP + 1], 1.0, None)
    lower = pooled[:, :DP] / cnt
    ub = ub_ref[0, 0]
    ub2 = ub2_ref[0, 0]
    gi = lax.broadcasted_iota(jnp.int32, (NU, NL), 0)
    uind = (gi == ub[None, :]).astype(jnp.float32)
    uind2 = (gi == ub2[None, :]).astype(jnp.float32)
    ucnt = jnp.clip(jnp.sum(uind, axis=1, keepdims=True), 1.0, None)
    out0 = jnp.dot(uind, lower, preferred_element_type=jnp.float32) / ucnt
    out1 = jnp.dot(uind2, lower, preferred_element_type=jnp.float32) / ucnt

    def classify(g):
        hc = jnp.dot(g, c1_ref[...], preferred_element_type=jnp.float32)
        hc = jnp.maximum(hc + c1b_ref[...], 0.0)
        return jnp.dot(hc, c2_ref[...],
                       preferred_element_type=jnp.float32) + c2b_ref[...]

    o_ref[...] = jnp.concatenate([classify(out0), classify(out1)], axis=0)


def _make_sc_layer(l):
    mesh = plsc.VectorSubcoreMesh(core_axis_name="c", subcore_axis_name="s")

    @functools.partial(
        pl.kernel,
        out_type=jax.ShapeDtypeStruct((2, N_OUT, DH), jnp.float32),
        scratch_types=[
            pltpu.VMEM((2, CH), jnp.int32),
            pltpu.VMEM((CH, DH), jnp.float32),
            pltpu.VMEM((CH, 256), jnp.float32),
            pltpu.VMEM_SHARED((N_ACC, DH), jnp.float32),
            pltpu.SemaphoreType.DMA,
        ],
        mesh=mesh,
        compiler_params=pltpu.CompilerParams(use_tc_tiling_on_sc=False),
    )
    def sc_layer(hflat, e_all, idx5, zeros, out,
                 idxc, hbuf, ebuf, acc, sem):
        c = lax.axis_index("c")
        s = lax.axis_index("s")
        pltpu.sync_copy(zeros.at[pl.ds(s * 626, 626)],
                        acc.at[pl.ds(s * 626, 626)])
        plsc.subcore_barrier()

        def chunk(j, carry):
            ebase = (s * CPW + j) * CH
            pltpu.sync_copy(idx5.at[c, s, j], idxc)
            pltpu.sync_copy(e_all.at[l, c, pl.ds(ebase, CH)], ebuf)
            pltpu.async_copy(hflat.at[idxc.at[0]], hbuf, sem).wait()

            def row(r, carry2):
                for k in range(DH // 16):
                    sl = pl.ds(k * 16, 16)
                    esl = pl.ds(k * 16 if k < 8 else 96 + k * 16, 16)
                    hbuf[r, sl] = jnp.maximum(hbuf[r, sl] + ebuf[r, esl], 0.0)
                return carry2

            lax.fori_loop(0, CH, row, 0)
            pltpu.sync_copy(hbuf, acc.at[idxc.at[1]], add=True)
            return carry

        lax.fori_loop(0, CPW, chunk, 0)
        plsc.subcore_barrier()
        pltpu.sync_copy(acc.at[pl.ds(s * 626, 626)],
                        out.at[c, pl.ds(s * 626, 626)])

    return sc_layer


def kernel(x, edge_index, edge_attr, lower_batch, upper_batch,
           W, b, Ew, Eb, C1, c1b, C2, c2b):
    f32 = jnp.float32
    src = edge_index[0]
    dst = edge_index[1]
    pad = E_PAD - E
    src_p = jnp.concatenate([src, jnp.zeros((pad,), jnp.int32)])
    dst_p = jnp.concatenate([dst, jnp.full((pad,), N, jnp.int32)])
    ea_p = jnp.concatenate([edge_attr, jnp.zeros((pad, DE), f32)], axis=0)
    src3 = src_p.reshape(NSUB, CPW, CH)
    dst3 = dst_p.reshape(NSUB, CPW, CH)
    idx5 = jnp.stack([jnp.stack([src3, dst3], axis=2),
                      jnp.stack([src3 + N, dst3], axis=2)])

    Ew_p = jnp.pad(Ew, ((0, 0), (0, 0), (0, DP - D)))
    Eb_p = jnp.pad(Eb, ((0, 0), (0, DP - D))).reshape(L, 1, DP)
    W_p = jnp.pad(W, ((0, 0), (0, DP - D), (0, DP - D)))
    b_p = jnp.pad(b, ((0, 0), (0, DP - D))).reshape(L, 1, DP)
    C1p = jnp.pad(C1, ((0, DP - D), (0, DP - D)))
    c1bp = jnp.pad(c1b, (0, DP - D)).reshape(1, DP)
    C2p = jnp.pad(C2, ((0, DP - D), (0, 127)))
    c2bp = jnp.pad(c2b, (0, 127)).reshape(1, 128)
    xp = jnp.pad(x, ((0, 0), (0, DP - D)))
    hflat = jnp.concatenate([xp[:, :DH], xp[:, DH:]], axis=0)
    zeros_acc = jnp.zeros((N_ACC, DH), f32)
    lb3 = lower_batch.reshape(N // BNP, 1, BNP)
    ub3 = upper_batch.reshape(1, 1, NL)
    ub23 = jnp.roll(upper_batch, -1).reshape(1, 1, NL)

    e_all = pl.pallas_call(
        _edge_proj_kernel,
        grid=(L, E_PAD // BE),
        in_specs=[
            pl.BlockSpec((BE, DE), lambda l_, i: (i, 0)),
            pl.BlockSpec((1, DE, DP), lambda l_, i: (l_, 0, 0)),
            pl.BlockSpec((1, 1, DP), lambda l_, i: (l_, 0, 0)),
        ],
        out_specs=pl.BlockSpec((1, 2, BE, 256), lambda l_, i: (l_, 0, i, 0)),
        out_shape=jax.ShapeDtypeStruct((L, 2, E_PAD, 256), f32),
    )(ea_p, Ew_p, Eb_p)

    dense = pl.pallas_call(
        _dense_kernel,
        grid=(N // BN,),
        in_specs=[
            pl.BlockSpec((2, BN, DH), lambda i: (0, i, 0)),
            pl.BlockSpec((DP, DP), lambda i: (0, 0)),
            pl.BlockSpec((1, DP), lambda i: (0, 0)),
        ],
        out_specs=pl.BlockSpec((2, BN, DH), lambda i: (0, i, 0)),
        out_shape=jax.ShapeDtypeStruct((2, N, DH), f32),
    )
    for l in range(L):
        agg2 = _make_sc_layer(l)(hflat, e_all, idx5, zeros_acc)
        h2 = dense(agg2, W_p[l], b_p[l])
        hflat = h2.reshape(2 * N, DH)

    pooled = pl.pallas_call(
        _lower_pool_kernel,
        grid=(N // BNP,),
        in_specs=[
            pl.BlockSpec((1, 1, BNP), lambda i: (i, 0, 0)),
            pl.BlockSpec((2, BNP, DH), lambda i: (0, i, 0)),
        ],
        out_specs=pl.BlockSpec((NL, DP + 8), lambda i: (0, 0)),
        out_shape=jax.ShapeDtypeStruct((NL, DP + 8), f32),
    )(lb3, h2)

    fin = pl.pallas_call(
        _final_kernel,
        in_specs=[
            pl.BlockSpec((NL, DP + 8), lambda: (0, 0)),
            pl.BlockSpec((1, 1, NL), lambda: (0, 0, 0)),
            pl.BlockSpec((1, 1, NL), lambda: (0, 0, 0)),
            pl.BlockSpec((DP, DP), lambda: (0, 0)),
            pl.BlockSpec((1, DP), lambda: (0, 0)),
            pl.BlockSpec((DP, 128), lambda: (0, 0)),
            pl.BlockSpec((1, 128), lambda: (0, 0)),
        ],
        out_specs=pl.BlockSpec((2 * NU, 128), lambda: (0, 0)),
        out_shape=jax.ShapeDtypeStruct((2 * NU, 128), f32),
    )(pooled, ub3, ub23, C1p, c1bp, C2p, c2bp)

    logits = fin[:, 0]
    labels = jnp.concatenate([jnp.zeros((NU,), f32), jnp.ones((NU,), f32)])
    return logits, labels


# speedup vs baseline: 1.0464x; 1.0464x over previous
import functools

import jax
import jax.numpy as jnp
from jax import lax
from jax.experimental import pallas as pl
from jax.experimental.pallas import tpu as pltpu
from jax.experimental.pallas import tpu_sc as plsc

N = 10000
E = 160000
D = 300
DE = 16
L = 5
NL = 2000
NU = 256

DP = 320
DH = DP // 2
NSUB = 16
CH = 64
CPW = 160
E_PAD = NSUB * CPW * CH
N_ACC = 10016
N_OUT = N_ACC
BN = 400
BNP = 1000
BE = 2048


def _edge_proj_kernel(ea_ref, ew_ref, eb_ref, oa_ref, oc_ref):
    v = jnp.dot(ea_ref[...], ew_ref[0], preferred_element_type=jnp.float32)
    v = v + eb_ref[0]
    oa_ref[0, 0] = v[:, 0:128]
    oa_ref[0, 1] = v[:, 160:288]
    oc_ref[0, 0] = v[:, 32:160]
    oc_ref[0, 1] = v[:, 192:320]


def _dense_kernel(a_ref, w_ref, b_ref, o_ref):
    a = jnp.concatenate([a_ref[0], a_ref[1]], axis=1)
    v = jnp.dot(a, w_ref[...], preferred_element_type=jnp.float32) + b_ref[...]
    v = jnp.maximum(v, 0.0)
    o_ref[0] = v[:, :DH]
    o_ref[1] = v[:, DH:]


def _lower_pool_kernel(lb_ref, h_ref, o_ref):
    i = pl.program_id(0)
    lb = lb_ref[0, 0]
    h = jnp.concatenate([h_ref[0], h_ref[1]], axis=1)
    haug = jnp.concatenate([h, jnp.ones((BNP, 8), jnp.float32)], axis=1)
    gi = lax.broadcasted_iota(jnp.int32, (NL, BNP), 0)
    ind = (gi == lb[None, :]).astype(jnp.float32)
    part = jnp.dot(ind, haug, preferred_element_type=jnp.float32)

    @pl.when(i == 0)
    def _():
        o_ref[...] = part

    @pl.when(i != 0)
    def _():
        o_ref[...] = o_ref[...] + part


def _final_kernel(p_ref, ub_ref, ub2_ref, c1_ref, c1b_ref, c2_ref, c2b_ref,
                  o_ref):
    pooled = p_ref[...]
    cnt = jnp.clip(pooled[:, DP:DP + 1], 1.0, None)
    lower = pooled[:, :DP] / cnt
    ub = ub_ref[0, 0]
    ub2 = ub2_ref[0, 0]
    gi = lax.broadcasted_iota(jnp.int32, (NU, NL), 0)
    uind = (gi == ub[None, :]).astype(jnp.float32)
    uind2 = (gi == ub2[None, :]).astype(jnp.float32)
    ucnt = jnp.clip(jnp.sum(uind, axis=1, keepdims=True), 1.0, None)
    out0 = jnp.dot(uind, lower, preferred_element_type=jnp.float32) / ucnt
    out1 = jnp.dot(uind2, lower, preferred_element_type=jnp.float32) / ucnt

    def classify(g):
        hc = jnp.dot(g, c1_ref[...], preferred_element_type=jnp.float32)
        hc = jnp.maximum(hc + c1b_ref[...], 0.0)
        return jnp.dot(hc, c2_ref[...],
                       preferred_element_type=jnp.float32) + c2b_ref[...]

    o_ref[...] = jnp.concatenate([classify(out0), classify(out1)], axis=0)


def _make_sc_layer(l):
    mesh = plsc.VectorSubcoreMesh(core_axis_name="c", subcore_axis_name="s")

    @functools.partial(
        pl.kernel,
        out_type=jax.ShapeDtypeStruct((2, N_OUT, DH), jnp.float32),
        scratch_types=[
            pltpu.VMEM((2, CH), jnp.int32),
            pltpu.VMEM((CH, DH), jnp.float32),
            pltpu.VMEM((CH, 128), jnp.float32),
            pltpu.VMEM((CH, 128), jnp.float32),
            pltpu.VMEM_SHARED((N_ACC, DH), jnp.float32),
            pltpu.SemaphoreType.DMA,
        ],
        mesh=mesh,
        compiler_params=pltpu.CompilerParams(use_tc_tiling_on_sc=False),
    )
    def sc_layer(hflat, ea, ec, idx5, zeros, out,
                 idxc, hbuf, ebufa, ebufc, acc, sem):
        c = lax.axis_index("c")
        s = lax.axis_index("s")
        pltpu.sync_copy(zeros.at[pl.ds(s * 626, 626)],
                        acc.at[pl.ds(s * 626, 626)])
        plsc.subcore_barrier()

        def chunk(j, carry):
            ebase = (s * CPW + j) * CH
            pltpu.sync_copy(idx5.at[c, s, j], idxc)
            pltpu.sync_copy(ea.at[l, c, pl.ds(ebase, CH)], ebufa)
            pltpu.sync_copy(ec.at[l, c, pl.ds(ebase, CH)], ebufc)
            pltpu.async_copy(hflat.at[idxc.at[0]], hbuf, sem).wait()

            def row(r, carry2):
                for k in range(8):
                    sl = pl.ds(k * 16, 16)
                    hbuf[r, sl] = jnp.maximum(hbuf[r, sl] + ebufa[r, sl], 0.0)
                for k in range(8, 10):
                    sl = pl.ds(k * 16, 16)
                    esl = pl.ds(96 + (k - 8) * 16, 16)
                    hbuf[r, sl] = jnp.maximum(hbuf[r, sl] + ebufc[r, esl], 0.0)
                return carry2

            lax.fori_loop(0, CH, row, 0)
            pltpu.sync_copy(hbuf, acc.at[idxc.at[1]], add=True)
            return carry

        lax.fori_loop(0, CPW, chunk, 0)
        plsc.subcore_barrier()
        pltpu.sync_copy(acc.at[pl.ds(s * 626, 626)],
                        out.at[c, pl.ds(s * 626, 626)])

    return sc_layer


def kernel(x, edge_index, edge_attr, lower_batch, upper_batch,
           W, b, Ew, Eb, C1, c1b, C2, c2b):
    f32 = jnp.float32
    src = edge_index[0]
    dst = edge_index[1]
    pad = E_PAD - E
    src_p = jnp.concatenate([src, jnp.zeros((pad,), jnp.int32)])
    dst_p = jnp.concatenate([dst, jnp.full((pad,), N, jnp.int32)])
    ea_p = jnp.concatenate([edge_attr, jnp.zeros((pad, DE), f32)], axis=0)
    src3 = src_p.reshape(NSUB, CPW, CH)
    dst3 = dst_p.reshape(NSUB, CPW, CH)
    idx5 = jnp.stack([jnp.stack([src3, dst3], axis=2),
                      jnp.stack([src3 + N, dst3], axis=2)])

    Ew_p = jnp.pad(Ew, ((0, 0), (0, 0), (0, DP - D)))
    Eb_p = jnp.pad(Eb, ((0, 0), (0, DP - D))).reshape(L, 1, DP)
    W_p = jnp.pad(W, ((0, 0), (0, DP - D), (0, DP - D)))
    b_p = jnp.pad(b, ((0, 0), (0, DP - D))).reshape(L, 1, DP)
    C1p = jnp.pad(C1, ((0, DP - D), (0, DP - D)))
    c1bp = jnp.pad(c1b, (0, DP - D)).reshape(1, DP)
    C2p = jnp.pad(C2, ((0, DP - D), (0, 127)))
    c2bp = jnp.pad(c2b, (0, 127)).reshape(1, 128)
    xp = jnp.pad(x, ((0, 0), (0, DP - D)))
    hflat = jnp.concatenate([xp[:, :DH], xp[:, DH:]], axis=0)
    zeros_acc = jnp.zeros((N_ACC, DH), f32)
    lb3 = lower_batch.reshape(N // BNP, 1, BNP)
    ub3 = upper_batch.reshape(1, 1, NL)
    ub23 = jnp.roll(upper_batch, -1).reshape(1, 1, NL)

    e_a, e_c = pl.pallas_call(
        _edge_proj_kernel,
        grid=(L, E_PAD // BE),
        in_specs=[
            pl.BlockSpec((BE, DE), lambda l_, i: (i, 0)),
            pl.BlockSpec((1, DE, DP), lambda l_, i: (l_, 0, 0)),
            pl.BlockSpec((1, 1, DP), lambda l_, i: (l_, 0, 0)),
        ],
        out_specs=[
            pl.BlockSpec((1, 2, BE, 128), lambda l_, i: (l_, 0, i, 0)),
            pl.BlockSpec((1, 2, BE, 128), lambda l_, i: (l_, 0, i, 0)),
        ],
        out_shape=[
            jax.ShapeDtypeStruct((L, 2, E_PAD, 128), f32),
            jax.ShapeDtypeStruct((L, 2, E_PAD, 128), f32),
        ],
    )(ea_p, Ew_p, Eb_p)

    dense = pl.pallas_call(
        _dense_kernel,
        grid=(N // BN,),
        in_specs=[
            pl.BlockSpec((2, BN, DH), lambda i: (0, i, 0)),
            pl.BlockSpec((DP, DP), lambda i: (0, 0)),
            pl.BlockSpec((1, DP), lambda i: (0, 0)),
        ],
        out_specs=pl.BlockSpec((2, BN, DH), lambda i: (0, i, 0)),
        out_shape=jax.ShapeDtypeStruct((2, N, DH), f32),
    )
    for l in range(L):
        agg2 = _make_sc_layer(l)(hflat, e_a, e_c, idx5, zeros_acc)
        h2 = dense(agg2, W_p[l], b_p[l])
        hflat = h2.reshape(2 * N, DH)

    pooled = pl.pallas_call(
        _lower_pool_kernel,
        grid=(N // BNP,),
        in_specs=[
            pl.BlockSpec((1, 1, BNP), lambda i: (i, 0, 0)),
            pl.BlockSpec((2, BNP, DH), lambda i: (0, i, 0)),
        ],
        out_specs=pl.BlockSpec((NL, DP + 8), lambda i: (0, 0)),
        out_shape=jax.ShapeDtypeStruct((NL, DP + 8), f32),
    )(lb3, h2)

    fin = pl.pallas_call(
        _final_kernel,
        in_specs=[
            pl.BlockSpec((NL, DP + 8), lambda: (0, 0)),
            pl.BlockSpec((1, 1, NL), lambda: (0, 0, 0)),
            pl.BlockSpec((1, 1, NL), lambda: (0, 0, 0)),
            pl.BlockSpec((DP, DP), lambda: (0, 0)),
            pl.BlockSpec((1, DP), lambda: (0, 0)),
            pl.BlockSpec((DP, 128), lambda: (0, 0)),
            pl.BlockSpec((1, 128), lambda: (0, 0)),
        ],
        out_specs=pl.BlockSpec((2 * NU, 128), lambda: (0, 0)),
        out_shape=jax.ShapeDtypeStruct((2 * NU, 128), f32),
    )(pooled, ub3, ub23, C1p, c1bp, C2p, c2bp)

    logits = fin[:, 0]
    labels = jnp.concatenate([jnp.zeros((NU,), f32), jnp.ones((NU,), f32)])
    return logits, labels


# speedup vs baseline: 1.8116x; 1.7313x over previous
import functools

import jax
import jax.numpy as jnp
from jax import lax
from jax.experimental import pallas as pl
from jax.experimental.pallas import tpu as pltpu
from jax.experimental.pallas import tpu_sc as plsc

N = 10000
E = 160000
D = 300
DE = 16
L = 5
NL = 2000
NU = 256

DP = 320
DH = DP // 2
NSUB = 16
CH = 24
CPW = 432
E_PAD = NSUB * CPW * CH
N_ACC = 10016
N_OUT = N_ACC
BN = 400
BNP = 1000
BE = 2048


def _edge_proj_kernel(ea_ref, ew_ref, eb_ref, oa_ref, oc_ref):
    v = jnp.dot(ea_ref[...], ew_ref[0], preferred_element_type=jnp.float32)
    v = v + eb_ref[0]
    oa_ref[0, 0] = v[:, 0:128]
    oa_ref[0, 1] = v[:, 160:288]
    oc_ref[0, 0] = v[:, 32:160]
    oc_ref[0, 1] = v[:, 192:320]


def _dense_kernel(a_ref, w_ref, b_ref, o_ref):
    a = jnp.concatenate([a_ref[0], a_ref[1]], axis=1)
    v = jnp.dot(a, w_ref[...], preferred_element_type=jnp.float32) + b_ref[...]
    v = jnp.maximum(v, 0.0)
    o_ref[0] = v[:, :DH]
    o_ref[1] = v[:, DH:]


def _lower_pool_kernel(lb_ref, h_ref, o_ref):
    i = pl.program_id(0)
    lb = lb_ref[0, 0]
    h = jnp.concatenate([h_ref[0], h_ref[1]], axis=1)
    haug = jnp.concatenate([h, jnp.ones((BNP, 8), jnp.float32)], axis=1)
    gi = lax.broadcasted_iota(jnp.int32, (NL, BNP), 0)
    ind = (gi == lb[None, :]).astype(jnp.float32)
    part = jnp.dot(ind, haug, preferred_element_type=jnp.float32)

    @pl.when(i == 0)
    def _():
        o_ref[...] = part

    @pl.when(i != 0)
    def _():
        o_ref[...] = o_ref[...] + part


def _final_kernel(p_ref, ub_ref, ub2_ref, c1_ref, c1b_ref, c2_ref, c2b_ref,
                  o_ref):
    pooled = p_ref[...]
    cnt = jnp.clip(pooled[:, DP:DP + 1], 1.0, None)
    lower = pooled[:, :DP] / cnt
    ub = ub_ref[0, 0]
    ub2 = ub2_ref[0, 0]
    gi = lax.broadcasted_iota(jnp.int32, (NU, NL), 0)
    uind = (gi == ub[None, :]).astype(jnp.float32)
    uind2 = (gi == ub2[None, :]).astype(jnp.float32)
    ucnt = jnp.clip(jnp.sum(uind, axis=1, keepdims=True), 1.0, None)
    out0 = jnp.dot(uind, lower, preferred_element_type=jnp.float32) / ucnt
    out1 = jnp.dot(uind2, lower, preferred_element_type=jnp.float32) / ucnt

    def classify(g):
        hc = jnp.dot(g, c1_ref[...], preferred_element_type=jnp.float32)
        hc = jnp.maximum(hc + c1b_ref[...], 0.0)
        return jnp.dot(hc, c2_ref[...],
                       preferred_element_type=jnp.float32) + c2b_ref[...]

    o_ref[...] = jnp.concatenate([classify(out0), classify(out1)], axis=0)


def _make_sc_layer(l):
    mesh = plsc.VectorSubcoreMesh(core_axis_name="c", subcore_axis_name="s")

    @functools.partial(
        pl.kernel,
        out_type=jax.ShapeDtypeStruct((2, N_OUT, DH), jnp.float32),
        scratch_types=[
            pltpu.VMEM((4, 2, CH), jnp.int32),
            pltpu.VMEM((4, CH, DH), jnp.float32),
            pltpu.VMEM((2, CH, 128), jnp.float32),
            pltpu.VMEM((2, CH, 128), jnp.float32),
            pltpu.VMEM_SHARED((N_ACC, DH), jnp.float32),
            pltpu.SemaphoreType.DMA((4,)),
            pltpu.SemaphoreType.DMA((4,)),
            pltpu.SemaphoreType.DMA((2,)),
            pltpu.SemaphoreType.DMA((2,)),
            pltpu.SemaphoreType.DMA((4,)),
        ],
        mesh=mesh,
        compiler_params=pltpu.CompilerParams(use_tc_tiling_on_sc=False),
    )
    def sc_layer(hflat, ea, ec, idx5, zeros, out,
                 idxc, hbuf, ebufa, ebufc, acc,
                 sem_i, sem_h, sem_a, sem_c, sem_s):
        c = lax.axis_index("c")
        s = lax.axis_index("s")
        pltpu.sync_copy(zeros.at[pl.ds(s * 626, 626)],
                        acc.at[pl.ds(s * 626, 626)])
        plsc.subcore_barrier()

        def issue_idx(j, p):
            pltpu.async_copy(idx5.at[c, s, j], idxc.at[p], sem_i.at[p])

        def wait_idx(p):
            pltpu.make_async_copy(idx5.at[c, s, 0], idxc.at[p],
                                  sem_i.at[p]).wait()

        def issue_data(j, p, pe):
            ebase = (s * CPW + j) * CH
            pltpu.async_copy(hflat.at[idxc.at[p, 0]], hbuf.at[p], sem_h.at[p])
            pltpu.async_copy(ea.at[l, c, pl.ds(ebase, CH)], ebufa.at[pe],
                             sem_a.at[pe])
            pltpu.async_copy(ec.at[l, c, pl.ds(ebase, CH)], ebufc.at[pe],
                             sem_c.at[pe])

        def wait_data(p, pe):
            pltpu.make_async_copy(hflat.at[idxc.at[p, 0]], hbuf.at[p],
                                  sem_h.at[p]).wait()
            pltpu.make_async_copy(ea.at[l, c, pl.ds(0, CH)], ebufa.at[pe],
                                  sem_a.at[pe]).wait()
            pltpu.make_async_copy(ec.at[l, c, pl.ds(0, CH)], ebufc.at[pe],
                                  sem_c.at[pe]).wait()

        def issue_scatter(p):
            pltpu.async_copy(hbuf.at[p], acc.at[idxc.at[p, 1]], sem_s.at[p],
                             add=True)

        def wait_scatter(p):
            pltpu.make_async_copy(hbuf.at[p], acc.at[idxc.at[p, 1]],
                                  sem_s.at[p]).wait()

        def compute(p, pe):
            def row(r, carry2):
                for k in range(8):
                    sl = pl.ds(k * 16, 16)
                    hbuf[p, r, sl] = jnp.maximum(
                        hbuf[p, r, sl] + ebufa[pe, r, sl], 0.0)
                for k in range(8, 10):
                    sl = pl.ds(k * 16, 16)
                    esl = pl.ds(96 + (k - 8) * 16, 16)
                    hbuf[p, r, sl] = jnp.maximum(
                        hbuf[p, r, sl] + ebufc[pe, r, esl], 0.0)
                return carry2

            lax.fori_loop(0, CH, row, 0)

        issue_idx(0, 0)
        issue_idx(1, 1)
        wait_idx(0)
        issue_data(0, 0, 0)

        def group(g, carry):
            for j0 in range(4):
                j = g * 4 + j0
                p1 = (j0 + 1) % 4
                p2 = (j0 + 2) % 4
                e0 = j0 % 2
                e1 = (j0 + 1) % 2

                @pl.when(j >= 2)
                def _():
                    wait_scatter(p2)

                @pl.when(j + 1 < CPW)
                def _():
                    wait_idx(p1)
                    issue_data(j + 1, p1, e1)

                @pl.when(j + 2 < CPW)
                def _():
                    issue_idx(j + 2, p2)

                wait_data(j0, e0)
                compute(j0, e0)
                issue_scatter(j0)
            return carry

        lax.fori_loop(0, CPW // 4, group, 0)
        wait_scatter((CPW - 2) % 4)
        wait_scatter((CPW - 1) % 4)
        plsc.subcore_barrier()
        pltpu.sync_copy(acc.at[pl.ds(s * 626, 626)],
                        out.at[c, pl.ds(s * 626, 626)])

    return sc_layer


def kernel(x, edge_index, edge_attr, lower_batch, upper_batch,
           W, b, Ew, Eb, C1, c1b, C2, c2b):
    f32 = jnp.float32
    src = edge_index[0]
    dst = edge_index[1]
    pad = E_PAD - E
    src_p = jnp.concatenate([src, jnp.zeros((pad,), jnp.int32)])
    dst_p = jnp.concatenate([dst, jnp.full((pad,), N, jnp.int32)])
    ea_p = jnp.concatenate([edge_attr, jnp.zeros((pad, DE), f32)], axis=0)
    src3 = src_p.reshape(NSUB, CPW, CH)
    dst3 = dst_p.reshape(NSUB, CPW, CH)
    idx5 = jnp.stack([jnp.stack([src3, dst3], axis=2),
                      jnp.stack([src3 + N, dst3], axis=2)])

    Ew_p = jnp.pad(Ew, ((0, 0), (0, 0), (0, DP - D)))
    Eb_p = jnp.pad(Eb, ((0, 0), (0, DP - D))).reshape(L, 1, DP)
    W_p = jnp.pad(W, ((0, 0), (0, DP - D), (0, DP - D)))
    b_p = jnp.pad(b, ((0, 0), (0, DP - D))).reshape(L, 1, DP)
    C1p = jnp.pad(C1, ((0, DP - D), (0, DP - D)))
    c1bp = jnp.pad(c1b, (0, DP - D)).reshape(1, DP)
    C2p = jnp.pad(C2, ((0, DP - D), (0, 127)))
    c2bp = jnp.pad(c2b, (0, 127)).reshape(1, 128)
    xp = jnp.pad(x, ((0, 0), (0, DP - D)))
    hflat = jnp.concatenate([xp[:, :DH], xp[:, DH:]], axis=0)
    zeros_acc = jnp.zeros((N_ACC, DH), f32)
    lb3 = lower_batch.reshape(N // BNP, 1, BNP)
    ub3 = upper_batch.reshape(1, 1, NL)
    ub23 = jnp.roll(upper_batch, -1).reshape(1, 1, NL)

    e_a, e_c = pl.pallas_call(
        _edge_proj_kernel,
        grid=(L, E_PAD // BE),
        in_specs=[
            pl.BlockSpec((BE, DE), lambda l_, i: (i, 0)),
            pl.BlockSpec((1, DE, DP), lambda l_, i: (l_, 0, 0)),
            pl.BlockSpec((1, 1, DP), lambda l_, i: (l_, 0, 0)),
        ],
        out_specs=[
            pl.BlockSpec((1, 2, BE, 128), lambda l_, i: (l_, 0, i, 0)),
            pl.BlockSpec((1, 2, BE, 128), lambda l_, i: (l_, 0, i, 0)),
        ],
        out_shape=[
            jax.ShapeDtypeStruct((L, 2, E_PAD, 128), f32),
            jax.ShapeDtypeStruct((L, 2, E_PAD, 128), f32),
        ],
    )(ea_p, Ew_p, Eb_p)

    dense = pl.pallas_call(
        _dense_kernel,
        grid=(N // BN,),
        in_specs=[
            pl.BlockSpec((2, BN, DH), lambda i: (0, i, 0)),
            pl.BlockSpec((DP, DP), lambda i: (0, 0)),
            pl.BlockSpec((1, DP), lambda i: (0, 0)),
        ],
        out_specs=pl.BlockSpec((2, BN, DH), lambda i: (0, i, 0)),
        out_shape=jax.ShapeDtypeStruct((2, N, DH), f32),
    )
    for l in range(L):
        agg2 = _make_sc_layer(l)(hflat, e_a, e_c, idx5, zeros_acc)
        h2 = dense(agg2, W_p[l], b_p[l])
        hflat = h2.reshape(2 * N, DH)

    pooled = pl.pallas_call(
        _lower_pool_kernel,
        grid=(N // BNP,),
        in_specs=[
            pl.BlockSpec((1, 1, BNP), lambda i: (i, 0, 0)),
            pl.BlockSpec((2, BNP, DH), lambda i: (0, i, 0)),
        ],
        out_specs=pl.BlockSpec((NL, DP + 8), lambda i: (0, 0)),
        out_shape=jax.ShapeDtypeStruct((NL, DP + 8), f32),
    )(lb3, h2)

    fin = pl.pallas_call(
        _final_kernel,
        in_specs=[
            pl.BlockSpec((NL, DP + 8), lambda: (0, 0)),
            pl.BlockSpec((1, 1, NL), lambda: (0, 0, 0)),
            pl.BlockSpec((1, 1, NL), lambda: (0, 0, 0)),
            pl.BlockSpec((DP, DP), lambda: (0, 0)),
            pl.BlockSpec((1, DP), lambda: (0, 0)),
            pl.BlockSpec((DP, 128), lambda: (0, 0)),
            pl.BlockSpec((1, 128), lambda: (0, 0)),
        ],
        out_specs=pl.BlockSpec((2 * NU, 128), lambda: (0, 0)),
        out_shape=jax.ShapeDtypeStruct((2 * NU, 128), f32),
    )(pooled, ub3, ub23, C1p, c1bp, C2p, c2bp)

    logits = fin[:, 0]
    labels = jnp.concatenate([jnp.zeros((NU,), f32), jnp.ones((NU,), f32)])
    return logits, labels


# speedup vs baseline: 1.9213x; 1.0606x over previous
import functools

import jax
import jax.numpy as jnp
from jax import lax
from jax.experimental import pallas as pl
from jax.experimental.pallas import tpu as pltpu
from jax.experimental.pallas import tpu_sc as plsc

N = 10000
E = 160000
D = 300
DE = 16
L = 5
NL = 2000
NU = 256

DP = 320
DH = DP // 2
NSUB = 16
CH = 24
CPW = 432
E_PAD = NSUB * CPW * CH
N_ACC = 10016
N_OUT = N_ACC
BN = 400
BNP = 1000
BE = 2048


def _edge_proj_kernel(ea_ref, ew_ref, eb_ref, oa_ref, oc_ref):
    v = jnp.dot(ea_ref[...], ew_ref[0], preferred_element_type=jnp.float32)
    v = v + eb_ref[0]
    oa_ref[0] = v[:, 0:128]
    oa_ref[1] = v[:, 160:288]
    oc_ref[0] = v[:, 32:160]
    oc_ref[1] = v[:, 192:320]


def _dense_kernel(a_ref, w_ref, b_ref, o_ref):
    a = jnp.concatenate([a_ref[0], a_ref[1]], axis=1)
    v = jnp.dot(a, w_ref[...], preferred_element_type=jnp.float32) + b_ref[...]
    v = jnp.maximum(v, 0.0)
    o_ref[0] = v[:, :DH]
    o_ref[1] = v[:, DH:]


def _lower_pool_kernel(lb_ref, h_ref, o_ref):
    i = pl.program_id(0)
    lb = lb_ref[0, 0]
    h = jnp.concatenate([h_ref[0], h_ref[1]], axis=1)
    haug = jnp.concatenate([h, jnp.ones((BNP, 8), jnp.float32)], axis=1)
    gi = lax.broadcasted_iota(jnp.int32, (NL, BNP), 0)
    ind = (gi == lb[None, :]).astype(jnp.float32)
    part = jnp.dot(ind, haug, preferred_element_type=jnp.float32)

    @pl.when(i == 0)
    def _():
        o_ref[...] = part

    @pl.when(i != 0)
    def _():
        o_ref[...] = o_ref[...] + part


def _final_kernel(p_ref, ub_ref, ub2_ref, c1_ref, c1b_ref, c2_ref, c2b_ref,
                  o_ref):
    pooled = p_ref[...]
    cnt = jnp.clip(pooled[:, DP:DP + 1], 1.0, None)
    lower = pooled[:, :DP] / cnt
    ub = ub_ref[0, 0]
    ub2 = ub2_ref[0, 0]
    gi = lax.broadcasted_iota(jnp.int32, (NU, NL), 0)
    uind = (gi == ub[None, :]).astype(jnp.float32)
    uind2 = (gi == ub2[None, :]).astype(jnp.float32)
    ucnt = jnp.clip(jnp.sum(uind, axis=1, keepdims=True), 1.0, None)
    out0 = jnp.dot(uind, lower, preferred_element_type=jnp.float32) / ucnt
    out1 = jnp.dot(uind2, lower, preferred_element_type=jnp.float32) / ucnt

    def classify(g):
        hc = jnp.dot(g, c1_ref[...], preferred_element_type=jnp.float32)
        hc = jnp.maximum(hc + c1b_ref[...], 0.0)
        return jnp.dot(hc, c2_ref[...],
                       preferred_element_type=jnp.float32) + c2b_ref[...]

    o_ref[...] = jnp.concatenate([classify(out0), classify(out1)], axis=0)


def _make_sc_layer():
    mesh = plsc.VectorSubcoreMesh(core_axis_name="c", subcore_axis_name="s")

    @functools.partial(
        pl.kernel,
        out_type=jax.ShapeDtypeStruct((2, N_OUT, DH), jnp.float32),
        scratch_types=[
            pltpu.VMEM((4, 2, CH), jnp.int32),
            pltpu.VMEM((4, CH, DH), jnp.float32),
            pltpu.VMEM((2, CH, 128), jnp.float32),
            pltpu.VMEM((2, CH, 128), jnp.float32),
            pltpu.VMEM_SHARED((N_ACC, DH), jnp.float32),
            pltpu.SemaphoreType.DMA((4,)),
            pltpu.SemaphoreType.DMA((4,)),
            pltpu.SemaphoreType.DMA((2,)),
            pltpu.SemaphoreType.DMA((2,)),
            pltpu.SemaphoreType.DMA((4,)),
        ],
        mesh=mesh,
        compiler_params=pltpu.CompilerParams(use_tc_tiling_on_sc=False),
    )
    def sc_layer(hflat, ea, ec, idx5, zeros, out,
                 idxc, hbuf, ebufa, ebufc, acc,
                 sem_i, sem_h, sem_a, sem_c, sem_s):
        c = lax.axis_index("c")
        s = lax.axis_index("s")
        pltpu.sync_copy(zeros.at[pl.ds(s * 626, 626)],
                        acc.at[pl.ds(s * 626, 626)])
        plsc.subcore_barrier()

        def issue_idx(j, p):
            pltpu.async_copy(idx5.at[c, s, j], idxc.at[p], sem_i.at[p])

        def wait_idx(p):
            pltpu.make_async_copy(idx5.at[c, s, 0], idxc.at[p],
                                  sem_i.at[p]).wait()

        def issue_data(j, p, pe):
            ebase = (s * CPW + j) * CH
            pltpu.async_copy(hflat.at[idxc.at[p, 0]], hbuf.at[p], sem_h.at[p])
            pltpu.async_copy(ea.at[c, pl.ds(ebase, CH)], ebufa.at[pe],
                             sem_a.at[pe])
            pltpu.async_copy(ec.at[c, pl.ds(ebase, CH)], ebufc.at[pe],
                             sem_c.at[pe])

        def wait_data(p, pe):
            pltpu.make_async_copy(hflat.at[idxc.at[p, 0]], hbuf.at[p],
                                  sem_h.at[p]).wait()
            pltpu.make_async_copy(ea.at[c, pl.ds(0, CH)], ebufa.at[pe],
                                  sem_a.at[pe]).wait()
            pltpu.make_async_copy(ec.at[c, pl.ds(0, CH)], ebufc.at[pe],
                                  sem_c.at[pe]).wait()

        def issue_scatter(p):
            pltpu.async_copy(hbuf.at[p], acc.at[idxc.at[p, 1]], sem_s.at[p],
                             add=True)

        def wait_scatter(p):
            pltpu.make_async_copy(hbuf.at[p], acc.at[idxc.at[p, 1]],
                                  sem_s.at[p]).wait()

        def compute(p, pe):
            def row(r, carry2):
                for k in range(8):
                    sl = pl.ds(k * 16, 16)
                    hbuf[p, r, sl] = jnp.maximum(
                        hbuf[p, r, sl] + ebufa[pe, r, sl], 0.0)
                for k in range(8, 10):
                    sl = pl.ds(k * 16, 16)
                    esl = pl.ds(96 + (k - 8) * 16, 16)
                    hbuf[p, r, sl] = jnp.maximum(
                        hbuf[p, r, sl] + ebufc[pe, r, esl], 0.0)
                return carry2

            lax.fori_loop(0, CH, row, 0)

        issue_idx(0, 0)
        issue_idx(1, 1)
        wait_idx(0)
        issue_data(0, 0, 0)

        def group(g, carry):
            for j0 in range(4):
                j = g * 4 + j0
                p1 = (j0 + 1) % 4
                p2 = (j0 + 2) % 4
                e0 = j0 % 2
                e1 = (j0 + 1) % 2

                @pl.when(j >= 2)
                def _():
                    wait_scatter(p2)

                @pl.when(j + 1 < CPW)
                def _():
                    wait_idx(p1)
                    issue_data(j + 1, p1, e1)

                @pl.when(j + 2 < CPW)
                def _():
                    issue_idx(j + 2, p2)

                wait_data(j0, e0)
                compute(j0, e0)
                issue_scatter(j0)
            return carry

        lax.fori_loop(0, CPW // 4, group, 0)
        wait_scatter((CPW - 2) % 4)
        wait_scatter((CPW - 1) % 4)
        plsc.subcore_barrier()
        pltpu.sync_copy(acc.at[pl.ds(s * 626, 626)],
                        out.at[c, pl.ds(s * 626, 626)])

    return sc_layer


def kernel(x, edge_index, edge_attr, lower_batch, upper_batch,
           W, b, Ew, Eb, C1, c1b, C2, c2b):
    f32 = jnp.float32
    src = edge_index[0]
    dst = edge_index[1]
    pad = E_PAD - E
    src_p = jnp.concatenate([src, jnp.zeros((pad,), jnp.int32)])
    dst_p = jnp.concatenate([dst, jnp.full((pad,), N, jnp.int32)])
    ea_p = jnp.concatenate([edge_attr, jnp.zeros((pad, DE), f32)], axis=0)
    src3 = src_p.reshape(NSUB, CPW, CH)
    dst3 = dst_p.reshape(NSUB, CPW, CH)
    idx5 = jnp.stack([jnp.stack([src3, dst3], axis=2),
                      jnp.stack([src3 + N, dst3], axis=2)])

    Ew_p = jnp.pad(Ew, ((0, 0), (0, 0), (0, DP - D)))
    Eb_p = jnp.pad(Eb, ((0, 0), (0, DP - D))).reshape(L, 1, DP)
    W_p = jnp.pad(W, ((0, 0), (0, DP - D), (0, DP - D)))
    b_p = jnp.pad(b, ((0, 0), (0, DP - D))).reshape(L, 1, DP)
    C1p = jnp.pad(C1, ((0, DP - D), (0, DP - D)))
    c1bp = jnp.pad(c1b, (0, DP - D)).reshape(1, DP)
    C2p = jnp.pad(C2, ((0, DP - D), (0, 127)))
    c2bp = jnp.pad(c2b, (0, 127)).reshape(1, 128)
    xp = jnp.pad(x, ((0, 0), (0, DP - D)))
    hflat = jnp.concatenate([xp[:, :DH], xp[:, DH:]], axis=0)
    zeros_acc = jnp.zeros((N_ACC, DH), f32)
    lb3 = lower_batch.reshape(N // BNP, 1, BNP)
    ub3 = upper_batch.reshape(1, 1, NL)
    ub23 = jnp.roll(upper_batch, -1).reshape(1, 1, NL)

    def edge_proj(li):
        return pl.pallas_call(
            _edge_proj_kernel,
            grid=(E_PAD // BE,),
            in_specs=[
                pl.BlockSpec((BE, DE), lambda i: (i, 0)),
                pl.BlockSpec((1, DE, DP), lambda i, li=li: (li, 0, 0)),
                pl.BlockSpec((1, 1, DP), lambda i, li=li: (li, 0, 0)),
            ],
            out_specs=[
                pl.BlockSpec((2, BE, 128), lambda i: (0, i, 0)),
                pl.BlockSpec((2, BE, 128), lambda i: (0, i, 0)),
            ],
            out_shape=[
                jax.ShapeDtypeStruct((2, E_PAD, 128), f32),
                jax.ShapeDtypeStruct((2, E_PAD, 128), f32),
            ],
        )(ea_p, Ew_p, Eb_p)

    dense = pl.pallas_call(
        _dense_kernel,
        grid=(N // BN,),
        in_specs=[
            pl.BlockSpec((2, BN, DH), lambda i: (0, i, 0)),
            pl.BlockSpec((DP, DP), lambda i: (0, 0)),
            pl.BlockSpec((1, DP), lambda i: (0, 0)),
        ],
        out_specs=pl.BlockSpec((2, BN, DH), lambda i: (0, i, 0)),
        out_shape=jax.ShapeDtypeStruct((2, N, DH), f32),
    )
    sc_layer = _make_sc_layer()
    e_keep = []
    for l in range(L):
        e_a, e_c = edge_proj(l)
        e_keep += [e_a, e_c]
        agg2 = sc_layer(hflat, e_a, e_c, idx5, zeros_acc)
        h2 = dense(agg2, W_p[l], b_p[l])
        hflat = h2.reshape(2 * N, DH)

    pooled = pl.pallas_call(
        _lower_pool_kernel,
        grid=(N // BNP,),
        in_specs=[
            pl.BlockSpec((1, 1, BNP), lambda i: (i, 0, 0)),
            pl.BlockSpec((2, BNP, DH), lambda i: (0, i, 0)),
        ],
        out_specs=pl.BlockSpec((NL, DP + 8), lambda i: (0, 0)),
        out_shape=jax.ShapeDtypeStruct((NL, DP + 8), f32),
    )(lb3, h2)

    fin = pl.pallas_call(
        _final_kernel,
        in_specs=[
            pl.BlockSpec((NL, DP + 8), lambda: (0, 0)),
            pl.BlockSpec((1, 1, NL), lambda: (0, 0, 0)),
            pl.BlockSpec((1, 1, NL), lambda: (0, 0, 0)),
            pl.BlockSpec((DP, DP), lambda: (0, 0)),
            pl.BlockSpec((1, DP), lambda: (0, 0)),
            pl.BlockSpec((DP, 128), lambda: (0, 0)),
            pl.BlockSpec((1, 128), lambda: (0, 0)),
        ],
        out_specs=pl.BlockSpec((2 * NU, 128), lambda: (0, 0)),
        out_shape=jax.ShapeDtypeStruct((2 * NU, 128), f32),
    )(pooled, ub3, ub23, C1p, c1bp, C2p, c2bp)

    fin, *_ = lax.optimization_barrier((fin, *e_keep))
    logits = fin[:, 0]
    labels = jnp.concatenate([jnp.zeros((NU,), f32), jnp.ones((NU,), f32)])
    return logits, labels


# speedup vs baseline: 2.1219x; 1.1044x over previous
import functools

import jax
import jax.numpy as jnp
from jax import lax
from jax.experimental import pallas as pl
from jax.experimental.pallas import tpu as pltpu
from jax.experimental.pallas import tpu_sc as plsc

N = 10000
E = 160000
D = 300
DE = 16
L = 5
NL = 2000
NU = 256

DP = 320
DH = DP // 2
NSUB = 16
CH = 24
CPW = 432
E_PAD = NSUB * CPW * CH
N_ACC = 10016
N_OUT = N_ACC
BN = 400
BNP = 1000
BE = 2048


def _edge_proj_kernel(ea_ref, ew_ref, eb_ref, oa_ref, oc_ref):
    v = jnp.dot(ea_ref[...], ew_ref[0], preferred_element_type=jnp.float32)
    v = v + eb_ref[0]
    oa_ref[0] = v[:, 0:128]
    oa_ref[1] = v[:, 160:288]
    oc_ref[0, :, 96:128] = v[:, 128:160]
    oc_ref[1, :, 96:128] = v[:, 288:320]


def _dense_kernel(a_ref, w_ref, b_ref, o_ref):
    a = jnp.concatenate([a_ref[0], a_ref[1]], axis=1)
    v = jnp.dot(a, w_ref[...], preferred_element_type=jnp.float32) + b_ref[...]
    v = jnp.maximum(v, 0.0)
    o_ref[0] = v[:, :DH]
    o_ref[1] = v[:, DH:]


def _lower_pool_kernel(lb_ref, h_ref, o_ref):
    i = pl.program_id(0)
    lb = lb_ref[0, 0]
    h = jnp.concatenate([h_ref[0], h_ref[1]], axis=1)
    haug = jnp.concatenate([h, jnp.ones((BNP, 8), jnp.float32)], axis=1)
    gi = lax.broadcasted_iota(jnp.int32, (NL, BNP), 0)
    ind = (gi == lb[None, :]).astype(jnp.float32)
    part = jnp.dot(ind, haug, preferred_element_type=jnp.float32)

    @pl.when(i == 0)
    def _():
        o_ref[...] = part

    @pl.when(i != 0)
    def _():
        o_ref[...] = o_ref[...] + part


def _final_kernel(p_ref, ub_ref, ub2_ref, c1_ref, c1b_ref, c2_ref, c2b_ref,
                  o_ref):
    pooled = p_ref[...]
    cnt = jnp.clip(pooled[:, DP:DP + 1], 1.0, None)
    lower = pooled[:, :DP] / cnt
    ub = ub_ref[0, 0]
    ub2 = ub2_ref[0, 0]
    gi = lax.broadcasted_iota(jnp.int32, (NU, NL), 0)
    uind = (gi == ub[None, :]).astype(jnp.float32)
    uind2 = (gi == ub2[None, :]).astype(jnp.float32)
    ucnt = jnp.clip(jnp.sum(uind, axis=1, keepdims=True), 1.0, None)
    out0 = jnp.dot(uind, lower, preferred_element_type=jnp.float32) / ucnt
    out1 = jnp.dot(uind2, lower, preferred_element_type=jnp.float32) / ucnt

    def classify(g):
        hc = jnp.dot(g, c1_ref[...], preferred_element_type=jnp.float32)
        hc = jnp.maximum(hc + c1b_ref[...], 0.0)
        return jnp.dot(hc, c2_ref[...],
                       preferred_element_type=jnp.float32) + c2b_ref[...]

    o_ref[...] = jnp.concatenate([classify(out0), classify(out1)], axis=0)


def _make_sc_layer():
    mesh = plsc.VectorSubcoreMesh(core_axis_name="c", subcore_axis_name="s")

    @functools.partial(
        pl.kernel,
        out_type=jax.ShapeDtypeStruct((2, N_OUT, DH), jnp.float32),
        scratch_types=[
            pltpu.VMEM((3, 4, 2, CH), jnp.int32),
            pltpu.VMEM((4, CH, DH), jnp.float32),
            pltpu.VMEM((2, CH, 128), jnp.float32),
            pltpu.VMEM((2, CH, 32), jnp.float32),
            pltpu.VMEM_SHARED((N_ACC, DH), jnp.float32),
            pltpu.SemaphoreType.DMA((3,)),
            pltpu.SemaphoreType.DMA((4,)),
            pltpu.SemaphoreType.DMA((2,)),
            pltpu.SemaphoreType.DMA((2,)),
            pltpu.SemaphoreType.DMA((4,)),
        ],
        mesh=mesh,
        compiler_params=pltpu.CompilerParams(use_tc_tiling_on_sc=False),
    )
    def sc_layer(hflat, ea, ec, idx6, zeros, out,
                 idxg, hbuf, ebufa, ebufc, acc,
                 sem_i, sem_h, sem_a, sem_c, sem_s):
        c = lax.axis_index("c")
        s = lax.axis_index("s")
        pltpu.sync_copy(zeros.at[pl.ds(s * 626, 626)],
                        acc.at[pl.ds(s * 626, 626)])
        plsc.subcore_barrier()

        NG = CPW // 4

        def issue_idxg(g, p):
            pltpu.async_copy(idx6.at[c, s, g], idxg.at[p], sem_i.at[p])

        def wait_idxg(p):
            pltpu.make_async_copy(idx6.at[c, s, 0], idxg.at[p],
                                  sem_i.at[p]).wait()

        def issue_data(j, p, pe, gs, gr):
            ebase = (s * CPW + j) * CH
            pltpu.async_copy(hflat.at[idxg.at[gs, gr, 0]], hbuf.at[p],
                             sem_h.at[p])
            pltpu.async_copy(ea.at[c, pl.ds(ebase, CH)], ebufa.at[pe],
                             sem_a.at[pe])
            pltpu.async_copy(ec.at[c, pl.ds(ebase, CH), pl.ds(96, 32)],
                             ebufc.at[pe], sem_c.at[pe])

        def wait_data(p, pe, gs, gr):
            pltpu.make_async_copy(hflat.at[idxg.at[gs, gr, 0]], hbuf.at[p],
                                  sem_h.at[p]).wait()
            pltpu.make_async_copy(ea.at[c, pl.ds(0, CH)], ebufa.at[pe],
                                  sem_a.at[pe]).wait()
            pltpu.make_async_copy(ec.at[c, pl.ds(0, CH), pl.ds(96, 32)],
                                  ebufc.at[pe], sem_c.at[pe]).wait()

        def issue_scatter(p, gs, gr):
            pltpu.async_copy(hbuf.at[p], acc.at[idxg.at[gs, gr, 1]],
                             sem_s.at[p], add=True)

        def wait_scatter(p, gs, gr):
            pltpu.make_async_copy(hbuf.at[p], acc.at[idxg.at[gs, gr, 1]],
                                  sem_s.at[p]).wait()

        def compute(p, pe):
            def row(r, carry2):
                for k in range(8):
                    sl = pl.ds(k * 16, 16)
                    hbuf[p, r, sl] = jnp.maximum(
                        hbuf[p, r, sl] + ebufa[pe, r, sl], 0.0)
                for k in range(8, 10):
                    sl = pl.ds(k * 16, 16)
                    esl = pl.ds((k - 8) * 16, 16)
                    hbuf[p, r, sl] = jnp.maximum(
                        hbuf[p, r, sl] + ebufc[pe, r, esl], 0.0)
                return carry2

            lax.fori_loop(0, CH, row, 0)

        issue_idxg(0, 0)
        issue_idxg(1, 1)
        wait_idxg(0)
        issue_data(0, 0, 0, 0, 0)

        def macro(m, carry):
            for gg in range(3):
                for j0 in range(4):
                    g = m * 3 + gg
                    j = g * 4 + j0
                    ngs = gg if j0 < 3 else (gg + 1) % 3
                    ngr = j0 + 1 if j0 < 3 else 0

                    @pl.when(j >= 2)
                    def _(j=j, j0=j0, gg=gg):
                        pgs = (gg + 2) % 3 if j0 < 2 else gg
                        wait_scatter((j0 + 2) % 4, pgs, (j0 + 2) % 4)

                    if j0 == 3:
                        @pl.when(j + 1 < CPW)
                        def _(gg=gg):
                            wait_idxg((gg + 1) % 3)

                    @pl.when(j + 1 < CPW)
                    def _(j=j, j0=j0, ngs=ngs, ngr=ngr):
                        issue_data(j + 1, (j0 + 1) % 4, (j0 + 1) % 2,
                                   ngs, ngr)

                    if j0 == 2:
                        @pl.when(g + 2 < NG)
                        def _(g=g, gg=gg):
                            issue_idxg(g + 2, (gg + 2) % 3)

                    wait_data(j0, j0 % 2, gg, j0)
                    compute(j0, j0 % 2)
                    issue_scatter(j0, gg, j0)
            return carry

        lax.fori_loop(0, CPW // 12, macro, 0)
        wait_scatter((CPW - 2) % 4, (NG - 1) % 3, 2)
        wait_scatter((CPW - 1) % 4, (NG - 1) % 3, 3)
        plsc.subcore_barrier()
        pltpu.sync_copy(acc.at[pl.ds(s * 626, 626)],
                        out.at[c, pl.ds(s * 626, 626)])

    return sc_layer


def kernel(x, edge_index, edge_attr, lower_batch, upper_batch,
           W, b, Ew, Eb, C1, c1b, C2, c2b):
    f32 = jnp.float32
    src = edge_index[0]
    dst = edge_index[1]
    pad = E_PAD - E
    src_p = jnp.concatenate([src, jnp.zeros((pad,), jnp.int32)])
    dst_p = jnp.concatenate([dst, jnp.full((pad,), N, jnp.int32)])
    ea_p = jnp.concatenate([edge_attr, jnp.zeros((pad, DE), f32)], axis=0)
    src3 = src_p.reshape(NSUB, CPW, CH)
    dst3 = dst_p.reshape(NSUB, CPW, CH)
    idx5 = jnp.stack([jnp.stack([src3, dst3], axis=2),
                      jnp.stack([src3 + N, dst3], axis=2)])
    idx6 = idx5.reshape(2, NSUB, CPW // 4, 4, 2, CH)

    Ew_p = jnp.pad(Ew, ((0, 0), (0, 0), (0, DP - D)))
    Eb_p = jnp.pad(Eb, ((0, 0), (0, DP - D))).reshape(L, 1, DP)
    W_p = jnp.pad(W, ((0, 0), (0, DP - D), (0, DP - D)))
    b_p = jnp.pad(b, ((0, 0), (0, DP - D))).reshape(L, 1, DP)
    C1p = jnp.pad(C1, ((0, DP - D), (0, DP - D)))
    c1bp = jnp.pad(c1b, (0, DP - D)).reshape(1, DP)
    C2p = jnp.pad(C2, ((0, DP - D), (0, 127)))
    c2bp = jnp.pad(c2b, (0, 127)).reshape(1, 128)
    xp = jnp.pad(x, ((0, 0), (0, DP - D)))
    hflat = jnp.concatenate([xp[:, :DH], xp[:, DH:]], axis=0)
    zeros_acc = jnp.zeros((N_ACC, DH), f32)
    lb3 = lower_batch.reshape(N // BNP, 1, BNP)
    ub3 = upper_batch.reshape(1, 1, NL)
    ub23 = jnp.roll(upper_batch, -1).reshape(1, 1, NL)

    def edge_proj(li):
        return pl.pallas_call(
            _edge_proj_kernel,
            grid=(E_PAD // BE,),
            in_specs=[
                pl.BlockSpec((BE, DE), lambda i: (i, 0)),
                pl.BlockSpec((1, DE, DP), lambda i, li=li: (li, 0, 0)),
                pl.BlockSpec((1, 1, DP), lambda i, li=li: (li, 0, 0)),
            ],
            out_specs=[
                pl.BlockSpec((2, BE, 128), lambda i: (0, i, 0)),
                pl.BlockSpec((2, BE, 128), lambda i: (0, i, 0)),
            ],
            out_shape=[
                jax.ShapeDtypeStruct((2, E_PAD, 128), f32),
                jax.ShapeDtypeStruct((2, E_PAD, 128), f32),
            ],
        )(ea_p, Ew_p, Eb_p)

    dense = pl.pallas_call(
        _dense_kernel,
        grid=(N // BN,),
        in_specs=[
            pl.BlockSpec((2, BN, DH), lambda i: (0, i, 0)),
            pl.BlockSpec((DP, DP), lambda i: (0, 0)),
            pl.BlockSpec((1, DP), lambda i: (0, 0)),
        ],
        out_specs=pl.BlockSpec((2, BN, DH), lambda i: (0, i, 0)),
        out_shape=jax.ShapeDtypeStruct((2, N, DH), f32),
    )
    sc_layer = _make_sc_layer()
    e_keep = []
    for l in range(L):
        e_a, e_c = edge_proj(l)
        e_keep += [e_a, e_c]
        agg2 = sc_layer(hflat, e_a, e_c, idx6, zeros_acc)
        h2 = dense(agg2, W_p[l], b_p[l])
        hflat = h2.reshape(2 * N, DH)

    pooled = pl.pallas_call(
        _lower_pool_kernel,
        grid=(N // BNP,),
        in_specs=[
            pl.BlockSpec((1, 1, BNP), lambda i: (i, 0, 0)),
            pl.BlockSpec((2, BNP, DH), lambda i: (0, i, 0)),
        ],
        out_specs=pl.BlockSpec((NL, DP + 8), lambda i: (0, 0)),
        out_shape=jax.ShapeDtypeStruct((NL, DP + 8), f32),
    )(lb3, h2)

    fin = pl.pallas_call(
        _final_kernel,
        in_specs=[
            pl.BlockSpec((NL, DP + 8), lambda: (0, 0)),
            pl.BlockSpec((1, 1, NL), lambda: (0, 0, 0)),
            pl.BlockSpec((1, 1, NL), lambda: (0, 0, 0)),
            pl.BlockSpec((DP, DP), lambda: (0, 0)),
            pl.BlockSpec((1, DP), lambda: (0, 0)),
            pl.BlockSpec((DP, 128), lambda: (0, 0)),
            pl.BlockSpec((1, 128), lambda: (0, 0)),
        ],
        out_specs=pl.BlockSpec((2 * NU, 128), lambda: (0, 0)),
        out_shape=jax.ShapeDtypeStruct((2 * NU, 128), f32),
    )(pooled, ub3, ub23, C1p, c1bp, C2p, c2bp)

    fin, *_ = lax.optimization_barrier((fin, *e_keep))
    logits = fin[:, 0]
    labels = jnp.concatenate([jnp.zeros((NU,), f32), jnp.ones((NU,), f32)])
    return logits, labels


# speedup vs baseline: 2.2974x; 1.0827x over previous
import functools

import jax
import jax.numpy as jnp
from jax import lax
from jax.experimental import pallas as pl
from jax.experimental.pallas import tpu as pltpu
from jax.experimental.pallas import tpu_sc as plsc

N = 10000
E = 160000
D = 300
DE = 16
L = 5
NL = 2000
NU = 256

DP = 320
DH = DP // 2
NSUB = 16
CH = 28
CPW = 360
E_PAD = NSUB * CPW * CH
N_ACC = 10016
N_OUT = N_ACC
BN = 400
BNP = 1000
BE = 1920


def _edge_proj_kernel(ea_ref, ew_ref, eb_ref, oa_ref, oc_ref):
    v = jnp.dot(ea_ref[...], ew_ref[0], preferred_element_type=jnp.float32)
    v = v + eb_ref[0]
    oa_ref[0] = v[:, 0:128]
    oa_ref[1] = v[:, 160:288]
    oc_ref[0, :, 96:128] = v[:, 128:160]
    oc_ref[1, :, 96:128] = v[:, 288:320]


def _dense_kernel(a_ref, w_ref, b_ref, o_ref):
    a = jnp.concatenate([a_ref[0], a_ref[1]], axis=1)
    v = jnp.dot(a, w_ref[...], preferred_element_type=jnp.float32) + b_ref[...]
    v = jnp.maximum(v, 0.0)
    o_ref[0] = v[:, :DH]
    o_ref[1] = v[:, DH:]


def _lower_pool_kernel(lb_ref, h_ref, o_ref):
    i = pl.program_id(0)
    lb = lb_ref[0, 0]
    h = jnp.concatenate([h_ref[0], h_ref[1]], axis=1)
    haug = jnp.concatenate([h, jnp.ones((BNP, 8), jnp.float32)], axis=1)
    gi = lax.broadcasted_iota(jnp.int32, (NL, BNP), 0)
    ind = (gi == lb[None, :]).astype(jnp.float32)
    part = jnp.dot(ind, haug, preferred_element_type=jnp.float32)

    @pl.when(i == 0)
    def _():
        o_ref[...] = part

    @pl.when(i != 0)
    def _():
        o_ref[...] = o_ref[...] + part


def _final_kernel(p_ref, ub_ref, ub2_ref, c1_ref, c1b_ref, c2_ref, c2b_ref,
                  o_ref):
    pooled = p_ref[...]
    cnt = jnp.clip(pooled[:, DP:DP + 1], 1.0, None)
    lower = pooled[:, :DP] / cnt
    ub = ub_ref[0, 0]
    ub2 = ub2_ref[0, 0]
    gi = lax.broadcasted_iota(jnp.int32, (NU, NL), 0)
    uind = (gi == ub[None, :]).astype(jnp.float32)
    uind2 = (gi == ub2[None, :]).astype(jnp.float32)
    ucnt = jnp.clip(jnp.sum(uind, axis=1, keepdims=True), 1.0, None)
    out0 = jnp.dot(uind, lower, preferred_element_type=jnp.float32) / ucnt
    out1 = jnp.dot(uind2, lower, preferred_element_type=jnp.float32) / ucnt

    def classify(g):
        hc = jnp.dot(g, c1_ref[...], preferred_element_type=jnp.float32)
        hc = jnp.maximum(hc + c1b_ref[...], 0.0)
        return jnp.dot(hc, c2_ref[...],
                       preferred_element_type=jnp.float32) + c2b_ref[...]

    o_ref[...] = jnp.concatenate([classify(out0), classify(out1)], axis=0)


def _make_sc_layer():
    mesh = plsc.VectorSubcoreMesh(core_axis_name="c", subcore_axis_name="s")

    @functools.partial(
        pl.kernel,
        out_type=jax.ShapeDtypeStruct((2, N_OUT, DH), jnp.float32),
        scratch_types=[
            pltpu.VMEM((3, 4, 2, CH), jnp.int32),
            pltpu.VMEM((4, CH, DH), jnp.float32),
            pltpu.VMEM((2, CH, 128), jnp.float32),
            pltpu.VMEM((2, CH, 32), jnp.float32),
            pltpu.VMEM_SHARED((N_ACC, DH), jnp.float32),
            pltpu.SemaphoreType.DMA((3,)),
            pltpu.SemaphoreType.DMA((4,)),
            pltpu.SemaphoreType.DMA((2,)),
            pltpu.SemaphoreType.DMA((2,)),
            pltpu.SemaphoreType.DMA((4,)),
        ],
        mesh=mesh,
        compiler_params=pltpu.CompilerParams(use_tc_tiling_on_sc=False),
    )
    def sc_layer(hflat, ea, ec, idx6, zeros, out,
                 idxg, hbuf, ebufa, ebufc, acc,
                 sem_i, sem_h, sem_a, sem_c, sem_s):
        c = lax.axis_index("c")
        s = lax.axis_index("s")
        pltpu.sync_copy(zeros.at[pl.ds(s * 626, 626)],
                        acc.at[pl.ds(s * 626, 626)])
        plsc.subcore_barrier()

        NG = CPW // 4

        def issue_idxg(g, p):
            pltpu.async_copy(idx6.at[c, s, g], idxg.at[p], sem_i.at[p])

        def wait_idxg(p):
            pltpu.make_async_copy(idx6.at[c, s, 0], idxg.at[p],
                                  sem_i.at[p]).wait()

        def issue_data(j, p, pe, gs, gr):
            ebase = (s * CPW + j) * CH
            pltpu.async_copy(hflat.at[idxg.at[gs, gr, 0]], hbuf.at[p],
                             sem_h.at[p])
            pltpu.async_copy(ea.at[c, pl.ds(ebase, CH)], ebufa.at[pe],
                             sem_a.at[pe])
            pltpu.async_copy(ec.at[c, pl.ds(ebase, CH), pl.ds(96, 32)],
                             ebufc.at[pe], sem_c.at[pe])

        def wait_data(p, pe, gs, gr):
            pltpu.make_async_copy(hflat.at[idxg.at[gs, gr, 0]], hbuf.at[p],
                                  sem_h.at[p]).wait()
            pltpu.make_async_copy(ea.at[c, pl.ds(0, CH)], ebufa.at[pe],
                                  sem_a.at[pe]).wait()
            pltpu.make_async_copy(ec.at[c, pl.ds(0, CH), pl.ds(96, 32)],
                                  ebufc.at[pe], sem_c.at[pe]).wait()

        def issue_scatter(p, gs, gr):
            pltpu.async_copy(hbuf.at[p], acc.at[idxg.at[gs, gr, 1]],
                             sem_s.at[p], add=True)

        def wait_scatter(p, gs, gr):
            pltpu.make_async_copy(hbuf.at[p], acc.at[idxg.at[gs, gr, 1]],
                                  sem_s.at[p]).wait()

        def compute(p, pe):
            def row4(r4, carry2):
                for u in range(4):
                    r = r4 * 4 + u
                    for k in range(8):
                        sl = pl.ds(k * 16, 16)
                        hbuf[p, r, sl] = jnp.maximum(
                            hbuf[p, r, sl] + ebufa[pe, r, sl], 0.0)
                    for k in range(8, 10):
                        sl = pl.ds(k * 16, 16)
                        esl = pl.ds((k - 8) * 16, 16)
                        hbuf[p, r, sl] = jnp.maximum(
                            hbuf[p, r, sl] + ebufc[pe, r, esl], 0.0)
                return carry2

            lax.fori_loop(0, CH // 4, row4, 0)

        issue_idxg(0, 0)
        issue_idxg(1, 1)
        wait_idxg(0)
        issue_data(0, 0, 0, 0, 0)

        def macro(m, carry):
            for gg in range(3):
                for j0 in range(4):
                    g = m * 3 + gg
                    j = g * 4 + j0
                    ngs = gg if j0 < 3 else (gg + 1) % 3
                    ngr = j0 + 1 if j0 < 3 else 0

                    @pl.when(j >= 2)
                    def _(j=j, j0=j0, gg=gg):
                        pgs = (gg + 2) % 3 if j0 < 2 else gg
                        wait_scatter((j0 + 2) % 4, pgs, (j0 + 2) % 4)

                    if j0 == 3:
                        @pl.when(j + 1 < CPW)
                        def _(gg=gg):
                            wait_idxg((gg + 1) % 3)

                    @pl.when(j + 1 < CPW)
                    def _(j=j, j0=j0, ngs=ngs, ngr=ngr):
                        issue_data(j + 1, (j0 + 1) % 4, (j0 + 1) % 2,
                                   ngs, ngr)

                    if j0 == 2:
                        @pl.when(g + 2 < NG)
                        def _(g=g, gg=gg):
                            issue_idxg(g + 2, (gg + 2) % 3)

                    wait_data(j0, j0 % 2, gg, j0)
                    compute(j0, j0 % 2)
                    issue_scatter(j0, gg, j0)
            return carry

        lax.fori_loop(0, CPW // 12, macro, 0)
        wait_scatter((CPW - 2) % 4, (NG - 1) % 3, 2)
        wait_scatter((CPW - 1) % 4, (NG - 1) % 3, 3)
        plsc.subcore_barrier()
        pltpu.sync_copy(acc.at[pl.ds(s * 626, 626)],
                        out.at[c, pl.ds(s * 626, 626)])

    return sc_layer


def kernel(x, edge_index, edge_attr, lower_batch, upper_batch,
           W, b, Ew, Eb, C1, c1b, C2, c2b):
    f32 = jnp.float32
    src = edge_index[0]
    dst = edge_index[1]
    pad = E_PAD - E
    src_p = jnp.concatenate([src, jnp.zeros((pad,), jnp.int32)])
    dst_p = jnp.concatenate([dst, jnp.full((pad,), N, jnp.int32)])
    ea_p = jnp.concatenate([edge_attr, jnp.zeros((pad, DE), f32)], axis=0)
    src3 = src_p.reshape(NSUB, CPW, CH)
    dst3 = dst_p.reshape(NSUB, CPW, CH)
    idx5 = jnp.stack([jnp.stack([src3, dst3], axis=2),
                      jnp.stack([src3 + N, dst3], axis=2)])
    idx6 = idx5.reshape(2, NSUB, CPW // 4, 4, 2, CH)

    Ew_p = jnp.pad(Ew, ((0, 0), (0, 0), (0, DP - D)))
    Eb_p = jnp.pad(Eb, ((0, 0), (0, DP - D))).reshape(L, 1, DP)
    W_p = jnp.pad(W, ((0, 0), (0, DP - D), (0, DP - D)))
    b_p = jnp.pad(b, ((0, 0), (0, DP - D))).reshape(L, 1, DP)
    C1p = jnp.pad(C1, ((0, DP - D), (0, DP - D)))
    c1bp = jnp.pad(c1b, (0, DP - D)).reshape(1, DP)
    C2p = jnp.pad(C2, ((0, DP - D), (0, 127)))
    c2bp = jnp.pad(c2b, (0, 127)).reshape(1, 128)
    xp = jnp.pad(x, ((0, 0), (0, DP - D)))
    hflat = jnp.concatenate([xp[:, :DH], xp[:, DH:]], axis=0)
    zeros_acc = jnp.zeros((N_ACC, DH), f32)
    lb3 = lower_batch.reshape(N // BNP, 1, BNP)
    ub3 = upper_batch.reshape(1, 1, NL)
    ub23 = jnp.roll(upper_batch, -1).reshape(1, 1, NL)

    def edge_proj(li):
        return pl.pallas_call(
            _edge_proj_kernel,
            grid=(E_PAD // BE,),
            in_specs=[
                pl.BlockSpec((BE, DE), lambda i: (i, 0)),
                pl.BlockSpec((1, DE, DP), lambda i, li=li: (li, 0, 0)),
                pl.BlockSpec((1, 1, DP), lambda i, li=li: (li, 0, 0)),
            ],
            out_specs=[
                pl.BlockSpec((2, BE, 128), lambda i: (0, i, 0)),
                pl.BlockSpec((2, BE, 128), lambda i: (0, i, 0)),
            ],
            out_shape=[
                jax.ShapeDtypeStruct((2, E_PAD, 128), f32),
                jax.ShapeDtypeStruct((2, E_PAD, 128), f32),
            ],
        )(ea_p, Ew_p, Eb_p)

    dense = pl.pallas_call(
        _dense_kernel,
        grid=(N // BN,),
        in_specs=[
            pl.BlockSpec((2, BN, DH), lambda i: (0, i, 0)),
            pl.BlockSpec((DP, DP), lambda i: (0, 0)),
            pl.BlockSpec((1, DP), lambda i: (0, 0)),
        ],
        out_specs=pl.BlockSpec((2, BN, DH), lambda i: (0, i, 0)),
        out_shape=jax.ShapeDtypeStruct((2, N, DH), f32),
    )
    sc_layer = _make_sc_layer()
    e_keep = []
    for l in range(L):
        e_a, e_c = edge_proj(l)
        e_keep += [e_a, e_c]
        agg2 = sc_layer(hflat, e_a, e_c, idx6, zeros_acc)
        h2 = dense(agg2, W_p[l], b_p[l])
        hflat = h2.reshape(2 * N, DH)

    pooled = pl.pallas_call(
        _lower_pool_kernel,
        grid=(N // BNP,),
        in_specs=[
            pl.BlockSpec((1, 1, BNP), lambda i: (i, 0, 0)),
            pl.BlockSpec((2, BNP, DH), lambda i: (0, i, 0)),
        ],
        out_specs=pl.BlockSpec((NL, DP + 8), lambda i: (0, 0)),
        out_shape=jax.ShapeDtypeStruct((NL, DP + 8), f32),
    )(lb3, h2)

    fin = pl.pallas_call(
        _final_kernel,
        in_specs=[
            pl.BlockSpec((NL, DP + 8), lambda: (0, 0)),
            pl.BlockSpec((1, 1, NL), lambda: (0, 0, 0)),
            pl.BlockSpec((1, 1, NL), lambda: (0, 0, 0)),
            pl.BlockSpec((DP, DP), lambda: (0, 0)),
            pl.BlockSpec((1, DP), lambda: (0, 0)),
            pl.BlockSpec((DP, 128), lambda: (0, 0)),
            pl.BlockSpec((1, 128), lambda: (0, 0)),
        ],
        out_specs=pl.BlockSpec((2 * NU, 128), lambda: (0, 0)),
        out_shape=jax.ShapeDtypeStruct((2 * NU, 128), f32),
    )(pooled, ub3, ub23, C1p, c1bp, C2p, c2bp)

    fin, *_ = lax.optimization_barrier((fin, *e_keep))
    logits = fin[:, 0]
    labels = jnp.concatenate([jnp.zeros((NU,), f32), jnp.ones((NU,), f32)])
    return logits, labels


# speedup vs baseline: 2.3565x; 1.0257x over previous
import functools

import jax
import jax.numpy as jnp
from jax import lax
from jax.experimental import pallas as pl
from jax.experimental.pallas import tpu as pltpu
from jax.experimental.pallas import tpu_sc as plsc

N = 10000
E = 160000
D = 300
DE = 16
L = 5
NL = 2000
NU = 256

DP = 320
DH = DP // 2
NSUB = 16
CH = 28
CPW = 360
E_PAD = NSUB * CPW * CH
N_ACC = 10016
N_OUT = N_ACC
BN = 400
BNP = 1000
BE = 1920


def _edge_proj_kernel(ea_ref, ew_ref, eb_ref, oa_ref, oc_ref):
    v = jnp.dot(ea_ref[...], ew_ref[0], preferred_element_type=jnp.float32)
    v = v + eb_ref[0]
    oa_ref[0] = v[:, 0:128]
    oa_ref[1] = v[:, 160:288]
    oc_ref[0, :, 96:128] = v[:, 128:160]
    oc_ref[1, :, 96:128] = v[:, 288:320]


def _dense_kernel(a_ref, w_ref, b_ref, o_ref):
    a = jnp.concatenate([a_ref[0], a_ref[1]], axis=1)
    v = jnp.dot(a, w_ref[...], preferred_element_type=jnp.float32) + b_ref[...]
    v = jnp.maximum(v, 0.0)
    o_ref[0] = v[:, :DH]
    o_ref[1] = v[:, DH:]


def _lower_pool_kernel(lb_ref, h_ref, o_ref):
    i = pl.program_id(0)
    lb = lb_ref[0, 0]
    h = jnp.concatenate([h_ref[0], h_ref[1]], axis=1)
    haug = jnp.concatenate([h, jnp.ones((BNP, 8), jnp.float32)], axis=1)
    gi = lax.broadcasted_iota(jnp.int32, (NL, BNP), 0)
    ind = (gi == lb[None, :]).astype(jnp.float32)
    part = jnp.dot(ind, haug, preferred_element_type=jnp.float32)

    @pl.when(i == 0)
    def _():
        o_ref[...] = part

    @pl.when(i != 0)
    def _():
        o_ref[...] = o_ref[...] + part


def _final_kernel(p_ref, ub_ref, ub2_ref, c1_ref, c1b_ref, c2_ref, c2b_ref,
                  o_ref):
    pooled = p_ref[...]
    cnt = jnp.clip(pooled[:, DP:DP + 1], 1.0, None)
    lower = pooled[:, :DP] / cnt
    ub = ub_ref[0, 0]
    ub2 = ub2_ref[0, 0]
    gi = lax.broadcasted_iota(jnp.int32, (NU, NL), 0)
    uind = (gi == ub[None, :]).astype(jnp.float32)
    uind2 = (gi == ub2[None, :]).astype(jnp.float32)
    ucnt = jnp.clip(jnp.sum(uind, axis=1, keepdims=True), 1.0, None)
    out0 = jnp.dot(uind, lower, preferred_element_type=jnp.float32) / ucnt
    out1 = jnp.dot(uind2, lower, preferred_element_type=jnp.float32) / ucnt

    def classify(g):
        hc = jnp.dot(g, c1_ref[...], preferred_element_type=jnp.float32)
        hc = jnp.maximum(hc + c1b_ref[...], 0.0)
        return jnp.dot(hc, c2_ref[...],
                       preferred_element_type=jnp.float32) + c2b_ref[...]

    o_ref[...] = jnp.concatenate([classify(out0), classify(out1)], axis=0)


def _make_sc_layer():
    mesh = plsc.VectorSubcoreMesh(core_axis_name="c", subcore_axis_name="s")

    @functools.partial(
        pl.kernel,
        out_type=jax.ShapeDtypeStruct((2, N_OUT, DH), jnp.float32),
        scratch_types=[
            pltpu.VMEM((3, 4, 2, CH), jnp.int32),
            pltpu.VMEM((4, CH, DH), jnp.float32),
            pltpu.VMEM((2, CH, 128), jnp.float32),
            pltpu.VMEM((2, CH, 32), jnp.float32),
            pltpu.VMEM_SHARED((N_ACC, DH), jnp.float32),
            pltpu.SemaphoreType.DMA((3,)),
            pltpu.SemaphoreType.DMA((4,)),
            pltpu.SemaphoreType.DMA((2,)),
            pltpu.SemaphoreType.DMA((2,)),
            pltpu.SemaphoreType.DMA((4,)),
        ],
        mesh=mesh,
        compiler_params=pltpu.CompilerParams(use_tc_tiling_on_sc=False),
    )
    def sc_layer(hflat, ea, ec, idx6, zeros, out,
                 idxg, hbuf, ebufa, ebufc, acc,
                 sem_i, sem_h, sem_a, sem_c, sem_s):
        c = lax.axis_index("c")
        s = lax.axis_index("s")
        pltpu.sync_copy(zeros.at[pl.ds(s * 626, 626)],
                        acc.at[pl.ds(s * 626, 626)])
        plsc.subcore_barrier()

        NG = CPW // 4

        def issue_idxg(g, p):
            pltpu.async_copy(idx6.at[c, s, g], idxg.at[p], sem_i.at[p])

        def wait_idxg(p):
            pltpu.make_async_copy(idx6.at[c, s, 0], idxg.at[p],
                                  sem_i.at[p]).wait()

        def issue_data(j, p, pe, gs, gr):
            ebase = (s * CPW + j) * CH
            pltpu.async_copy(hflat.at[idxg.at[gs, gr, 0]], hbuf.at[p],
                             sem_h.at[p])
            pltpu.async_copy(ea.at[c, pl.ds(ebase, CH)], ebufa.at[pe],
                             sem_a.at[pe])
            pltpu.async_copy(ec.at[c, pl.ds(ebase, CH), pl.ds(96, 32)],
                             ebufc.at[pe], sem_c.at[pe])

        def wait_data(p, pe, gs, gr):
            pltpu.make_async_copy(hflat.at[idxg.at[gs, gr, 0]], hbuf.at[p],
                                  sem_h.at[p]).wait()
            pltpu.make_async_copy(ea.at[c, pl.ds(0, CH)], ebufa.at[pe],
                                  sem_a.at[pe]).wait()
            pltpu.make_async_copy(ec.at[c, pl.ds(0, CH), pl.ds(96, 32)],
                                  ebufc.at[pe], sem_c.at[pe]).wait()

        def issue_scatter(p, gs, gr):
            pltpu.async_copy(hbuf.at[p], acc.at[idxg.at[gs, gr, 1]],
                             sem_s.at[p], add=True)

        def wait_scatter(p, gs, gr):
            pltpu.make_async_copy(hbuf.at[p], acc.at[idxg.at[gs, gr, 1]],
                                  sem_s.at[p]).wait()

        def compute(p, pe):
            def row4(r4, carry2):
                for u in range(4):
                    r = r4 * 4 + u
                    for k in range(8):
                        sl = pl.ds(k * 16, 16)
                        hbuf[p, r, sl] = jnp.maximum(
                            hbuf[p, r, sl] + ebufa[pe, r, sl], 0.0)
                    for k in range(8, 10):
                        sl = pl.ds(k * 16, 16)
                        esl = pl.ds((k - 8) * 16, 16)
                        hbuf[p, r, sl] = jnp.maximum(
                            hbuf[p, r, sl] + ebufc[pe, r, esl], 0.0)
                return carry2

            lax.fori_loop(0, CH // 4, row4, 0)

        issue_idxg(0, 0)
        issue_idxg(1, 1)
        wait_idxg(0)
        issue_data(0, 0, 0, 0, 0)

        def macro(m, carry):
            for gg in range(3):
                for j0 in range(4):
                    g = m * 3 + gg
                    j = g * 4 + j0
                    ngs = gg if j0 < 3 else (gg + 1) % 3
                    ngr = j0 + 1 if j0 < 3 else 0

                    @pl.when(j >= 2)
                    def _(j=j, j0=j0, gg=gg):
                        pgs = (gg + 2) % 3 if j0 < 2 else gg
                        wait_scatter((j0 + 2) % 4, pgs, (j0 + 2) % 4)

                    if j0 == 3:
                        @pl.when(j + 1 < CPW)
                        def _(gg=gg):
                            wait_idxg((gg + 1) % 3)

                    @pl.when(j + 1 < CPW)
                    def _(j=j, j0=j0, ngs=ngs, ngr=ngr):
                        issue_data(j + 1, (j0 + 1) % 4, (j0 + 1) % 2,
                                   ngs, ngr)

                    if j0 == 2:
                        @pl.when(g + 2 < NG)
                        def _(g=g, gg=gg):
                            issue_idxg(g + 2, (gg + 2) % 3)

                    wait_data(j0, j0 % 2, gg, j0)
                    compute(j0, j0 % 2)
                    issue_scatter(j0, gg, j0)
            return carry

        lax.fori_loop(0, CPW // 12, macro, 0)
        wait_scatter((CPW - 2) % 4, (NG - 1) % 3, 2)
        wait_scatter((CPW - 1) % 4, (NG - 1) % 3, 3)
        plsc.subcore_barrier()
        pltpu.sync_copy(acc.at[pl.ds(s * 626, 626)],
                        out.at[c, pl.ds(s * 626, 626)])

    return sc_layer


def kernel(x, edge_index, edge_attr, lower_batch, upper_batch,
           W, b, Ew, Eb, C1, c1b, C2, c2b):
    f32 = jnp.float32
    src = edge_index[0]
    dst = edge_index[1]
    pad = E_PAD - E
    pad_ar = jnp.arange(pad, dtype=jnp.int32)
    src_p = jnp.concatenate([src, pad_ar % N])
    dst_p = jnp.concatenate([dst, N + (pad_ar % (N_ACC - N))])
    ea_p = jnp.concatenate([edge_attr, jnp.zeros((pad, DE), f32)], axis=0)
    src3 = src_p.reshape(NSUB, CPW, CH)
    dst3 = dst_p.reshape(NSUB, CPW, CH)
    idx5 = jnp.stack([jnp.stack([src3, dst3], axis=2),
                      jnp.stack([src3 + N, dst3], axis=2)])
    idx6 = idx5.reshape(2, NSUB, CPW // 4, 4, 2, CH)

    Ew_p = jnp.pad(Ew, ((0, 0), (0, 0), (0, DP - D)))
    Eb_p = jnp.pad(Eb, ((0, 0), (0, DP - D))).reshape(L, 1, DP)
    W_p = jnp.pad(W, ((0, 0), (0, DP - D), (0, DP - D)))
    b_p = jnp.pad(b, ((0, 0), (0, DP - D))).reshape(L, 1, DP)
    C1p = jnp.pad(C1, ((0, DP - D), (0, DP - D)))
    c1bp = jnp.pad(c1b, (0, DP - D)).reshape(1, DP)
    C2p = jnp.pad(C2, ((0, DP - D), (0, 127)))
    c2bp = jnp.pad(c2b, (0, 127)).reshape(1, 128)
    xp = jnp.pad(x, ((0, 0), (0, DP - D)))
    hflat = jnp.concatenate([xp[:, :DH], xp[:, DH:]], axis=0)
    zeros_acc = jnp.zeros((N_ACC, DH), f32)
    lb3 = lower_batch.reshape(N // BNP, 1, BNP)
    ub3 = upper_batch.reshape(1, 1, NL)
    ub23 = jnp.roll(upper_batch, -1).reshape(1, 1, NL)

    def edge_proj(li):
        return pl.pallas_call(
            _edge_proj_kernel,
            grid=(E_PAD // BE,),
            in_specs=[
                pl.BlockSpec((BE, DE), lambda i: (i, 0)),
                pl.BlockSpec((1, DE, DP), lambda i, li=li: (li, 0, 0)),
                pl.BlockSpec((1, 1, DP), lambda i, li=li: (li, 0, 0)),
            ],
            out_specs=[
                pl.BlockSpec((2, BE, 128), lambda i: (0, i, 0)),
                pl.BlockSpec((2, BE, 128), lambda i: (0, i, 0)),
            ],
            out_shape=[
                jax.ShapeDtypeStruct((2, E_PAD, 128), f32),
                jax.ShapeDtypeStruct((2, E_PAD, 128), f32),
            ],
        )(ea_p, Ew_p, Eb_p)

    dense = pl.pallas_call(
        _dense_kernel,
        grid=(N // BN,),
        in_specs=[
            pl.BlockSpec((2, BN, DH), lambda i: (0, i, 0)),
            pl.BlockSpec((DP, DP), lambda i: (0, 0)),
            pl.BlockSpec((1, DP), lambda i: (0, 0)),
        ],
        out_specs=pl.BlockSpec((2, BN, DH), lambda i: (0, i, 0)),
        out_shape=jax.ShapeDtypeStruct((2, N, DH), f32),
    )
    sc_layer = _make_sc_layer()
    e_keep = []
    for l in range(L):
        e_a, e_c = edge_proj(l)
        e_keep += [e_a, e_c]
        agg2 = sc_layer(hflat, e_a, e_c, idx6, zeros_acc)
        h2 = dense(agg2, W_p[l], b_p[l])
        hflat = h2.reshape(2 * N, DH)

    pooled = pl.pallas_call(
        _lower_pool_kernel,
        grid=(N // BNP,),
        in_specs=[
            pl.BlockSpec((1, 1, BNP), lambda i: (i, 0, 0)),
            pl.BlockSpec((2, BNP, DH), lambda i: (0, i, 0)),
        ],
        out_specs=pl.BlockSpec((NL, DP + 8), lambda i: (0, 0)),
        out_shape=jax.ShapeDtypeStruct((NL, DP + 8), f32),
    )(lb3, h2)

    fin = pl.pallas_call(
        _final_kernel,
        in_specs=[
            pl.BlockSpec((NL, DP + 8), lambda: (0, 0)),
            pl.BlockSpec((1, 1, NL), lambda: (0, 0, 0)),
            pl.BlockSpec((1, 1, NL), lambda: (0, 0, 0)),
            pl.BlockSpec((DP, DP), lambda: (0, 0)),
            pl.BlockSpec((1, DP), lambda: (0, 0)),
            pl.BlockSpec((DP, 128), lambda: (0, 0)),
            pl.BlockSpec((1, 128), lambda: (0, 0)),
        ],
        out_specs=pl.BlockSpec((2 * NU, 128), lambda: (0, 0)),
        out_shape=jax.ShapeDtypeStruct((2 * NU, 128), f32),
    )(pooled, ub3, ub23, C1p, c1bp, C2p, c2bp)

    fin, *_ = lax.optimization_barrier((fin, *e_keep))
    logits = fin[:, 0]
    labels = jnp.concatenate([jnp.zeros((NU,), f32), jnp.ones((NU,), f32)])
    return logits, labels
